# baseline (speedup 1.0000x reference)
import jax
import jax.numpy as jnp
from jax import lax
from jax.experimental import pallas as pl
from jax.experimental.pallas import tpu as pltpu
from jax.experimental.pallas import tpu_sc as plsc

_NUM_TYPES = 16
_HIDDEN = 512
_EMBED = 768
_B = 16384

_NC = 2
_NS = 16
_NW = _NC * _NS
_BPW = _B // _NW
_LANES = 16
_CHUNK = _LANES
_NCHUNK = _BPW // _CHUNK
_NBUF = 6


def _mlp_body(table_ref, w1_ref, b1_ref, w2_ref, b2_ref, out_ref):
    h = jnp.dot(table_ref[...], w1_ref[...], preferred_element_type=jnp.float32)
    h = h + b1_ref[...]
    h = h * jax.nn.sigmoid(h)
    out_ref[...] = (
        jnp.dot(h, w2_ref[...], preferred_element_type=jnp.float32) + b2_ref[...]
    )


def _mlp_table(table, W1, b1, W2, b2):
    return pl.pallas_call(
        _mlp_body,
        out_shape=jax.ShapeDtypeStruct((_NUM_TYPES, _EMBED), jnp.float32),
    )(table, W1, b1.reshape(1, _HIDDEN), W2, b2.reshape(1, _EMBED))


def _gather_body(tab_hbm, idx_hbm, out_hbm, tab_v, idx_v, *bufs):
    rows = bufs[:_NBUF]
    ssem = bufs[_NBUF:]

    wid = lax.axis_index("s") * _NC + lax.axis_index("c")
    base = wid * _BPW
    pltpu.sync_copy(tab_hbm, tab_v)
    pltpu.sync_copy(idx_hbm.at[pl.ds(base, _BPW)], idx_v)

    col_iota = lax.broadcasted_iota(jnp.int32, (_LANES,), 0)
    dst_base = col_iota * _EMBED

    def dst(c):
        return out_hbm.at[pl.ds((base + c * _CHUNK) * _EMBED, _CHUNK * _EMBED)]

    def sstart(c):
        pltpu.async_copy(rows[c % _NBUF], dst(c), ssem[c % _NBUF])

    def swait(c):
        pltpu.make_async_copy(rows[c % _NBUF], dst(c), ssem[c % _NBUF]).wait()

    def compute_group(g, buf):
        idx16 = idx_v[pl.ds(g * _LANES, _LANES)]
        src_base = idx16 * _EMBED

        @plsc.parallel_loop(0, _EMBED, step=1, unroll=8)
        def _(k):
            vals = plsc.load_gather(tab_v, [src_base + k])
            plsc.store_scatter(buf, [dst_base + k], vals)

    for g in range(_NCHUNK):
        if g >= _NBUF:
            swait(g - _NBUF)
        compute_group(g, rows[g % _NBUF])
        sstart(g)
    for g in range(_NCHUNK - _NBUF, _NCHUNK):
        swait(g)


def _gather(out_table_flat, tok):
    mesh = plsc.VectorSubcoreMesh(
        core_axis_name="c", subcore_axis_name="s", num_cores=_NC
    )
    run = pl.kernel(
        _gather_body,
        out_type=jax.ShapeDtypeStruct((_B * _EMBED,), jnp.float32),
        mesh=mesh,
        compiler_params=pltpu.CompilerParams(needs_layout_passes=False),
        scratch_types=(
            [
                pltpu.VMEM((_NUM_TYPES * _EMBED,), jnp.float32),
                pltpu.VMEM((_BPW,), jnp.int32),
            ]
            + [pltpu.VMEM((_CHUNK * _EMBED,), jnp.float32) for _ in range(_NBUF)]
            + [pltpu.SemaphoreType.DMA for _ in range(_NBUF)]
        ),
    )
    return run(out_table_flat, tok)


def kernel(goal_tokens, table, W1, b1, W2, b2):
    tok = goal_tokens.astype(jnp.int32)
    out_table = _mlp_table(table, W1, b1, W2, b2)
    out_flat = _gather(out_table.reshape(-1), tok)
    return out_flat.reshape(_B, _EMBED)


# speedup vs baseline: 3.2397x; 3.2397x over previous
import jax
import jax.numpy as jnp
from jax import lax
from jax.experimental import pallas as pl
from jax.experimental.pallas import tpu as pltpu
from jax.experimental.pallas import tpu_sc as plsc

_NUM_TYPES = 16
_HIDDEN = 512
_EMBED = 768
_B = 16384

_NC = 2
_NS = 16
_NW = _NC * _NS
_BPW = _B // _NW
_LANES = 16
_CHUNK = _LANES
_NCHUNK = _BPW // _CHUNK
_NBUF = 6


def _mlp_body(table_ref, w1_ref, b1_ref, w2_ref, b2_ref, out_ref):
    h = jnp.dot(table_ref[...], w1_ref[...], preferred_element_type=jnp.float32)
    h = h + b1_ref[...]
    h = h * jax.nn.sigmoid(h)
    out_ref[...] = (
        jnp.dot(h, w2_ref[...], preferred_element_type=jnp.float32) + b2_ref[...]
    )


def _mlp_table(table, W1, b1, W2, b2):
    return pl.pallas_call(
        _mlp_body,
        out_shape=jax.ShapeDtypeStruct((_NUM_TYPES, _EMBED), jnp.float32),
    )(table, W1, b1.reshape(1, _HIDDEN), W2, b2.reshape(1, _EMBED))


def _gather_body(tab_hbm, idx_hbm, out_hbm, tab_v, idx_v, *bufs):
    rows = bufs[:_NBUF]
    ssem = bufs[_NBUF:]

    wid = lax.axis_index("s") * _NC + lax.axis_index("c")
    base = wid * _BPW
    pltpu.sync_copy(tab_hbm, tab_v)
    pltpu.sync_copy(idx_hbm.at[pl.ds(base, _BPW)], idx_v)

    col_iota = lax.broadcasted_iota(jnp.int32, (_LANES,), 0)

    def dst(c):
        return out_hbm.at[pl.ds((base + c * _CHUNK) * _EMBED, _CHUNK * _EMBED)]

    def sstart(c):
        pltpu.async_copy(rows[c % _NBUF], dst(c), ssem[c % _NBUF])

    def swait(c):
        pltpu.make_async_copy(rows[c % _NBUF], dst(c), ssem[c % _NBUF]).wait()

    def compute_group(g, buf):
        idx16 = idx_v[pl.ds(g * _LANES, _LANES)]
        src_base = idx16 * _EMBED

        def row_body(j, carry):
            lane = jnp.full((_LANES,), j, dtype=jnp.int32)
            rowb = jnp.take_along_axis(src_base, lane, axis=0)

            @plsc.parallel_loop(0, _EMBED // _LANES, step=1, unroll=4)
            def _(c):
                ptr = rowb + (col_iota + c * _LANES)
                vals = plsc.load_gather(tab_v, [ptr])
                buf[pl.ds(j * _EMBED + c * _LANES, _LANES)] = vals

            return carry

        lax.fori_loop(0, _LANES, row_body, 0)

    for g in range(_NCHUNK):
        if g >= _NBUF:
            swait(g - _NBUF)
        compute_group(g, rows[g % _NBUF])
        sstart(g)
    for g in range(_NCHUNK - _NBUF, _NCHUNK):
        swait(g)


def _gather(out_table_flat, tok):
    mesh = plsc.VectorSubcoreMesh(
        core_axis_name="c", subcore_axis_name="s", num_cores=_NC
    )
    run = pl.kernel(
        _gather_body,
        out_type=jax.ShapeDtypeStruct((_B * _EMBED,), jnp.float32),
        mesh=mesh,
        compiler_params=pltpu.CompilerParams(needs_layout_passes=False),
        scratch_types=(
            [
                pltpu.VMEM((_NUM_TYPES * _EMBED,), jnp.float32),
                pltpu.VMEM((_BPW,), jnp.int32),
            ]
            + [pltpu.VMEM((_CHUNK * _EMBED,), jnp.float32) for _ in range(_NBUF)]
            + [pltpu.SemaphoreType.DMA for _ in range(_NBUF)]
        ),
    )
    return run(out_table_flat, tok)


def kernel(goal_tokens, table, W1, b1, W2, b2):
    tok = goal_tokens.astype(jnp.int32)
    out_table = _mlp_table(table, W1, b1, W2, b2)
    out_flat = _gather(out_table.reshape(-1), tok)
    return out_flat.reshape(_B, _EMBED)


# speedup vs baseline: 5.8468x; 1.8047x over previous
import jax
import jax.numpy as jnp
from jax import lax
from jax.experimental import pallas as pl
from jax.experimental.pallas import tpu as pltpu
from jax.experimental.pallas import tpu_sc as plsc

_NUM_TYPES = 16
_HIDDEN = 512
_EMBED = 768
_B = 16384

_NC = 2
_NS = 16
_NW = _NC * _NS
_BPW = _B // _NW
_NBUF = 8
_LOOK = 3
_CHUNK = 16
_NCHUNK = _BPW // _CHUNK
_LANES = 16
_REP_PER_TILE = 4
_NREP = _NW * _REP_PER_TILE


def _mlp_body(table_ref, w1_ref, b1_ref, w2_ref, b2_ref, out_ref):
    h = jnp.dot(table_ref[...], w1_ref[...], preferred_element_type=jnp.float32)
    h = h + b1_ref[...]
    h = h * jax.nn.sigmoid(h)
    o = jnp.dot(h, w2_ref[...], preferred_element_type=jnp.float32) + b2_ref[...]
    for r in range(_NREP):
        out_ref[pl.ds(r * _NUM_TYPES, _NUM_TYPES), :] = o


def _mlp_table(table, W1, b1, W2, b2):
    return pl.pallas_call(
        _mlp_body,
        out_shape=jax.ShapeDtypeStruct((_NREP * _NUM_TYPES, _EMBED), jnp.float32),
    )(table, W1, b1.reshape(1, _HIDDEN), W2, b2.reshape(1, _EMBED))


def _gather_body(tab_hbm, idx_hbm, out_hbm, idx_v, *bufs):
    rows = bufs[:_NBUF]
    gsem = bufs[_NBUF : 2 * _NBUF]
    ssem = bufs[2 * _NBUF :]

    wid = lax.axis_index("s") * _NC + lax.axis_index("c")
    base = wid * _BPW
    pltpu.sync_copy(idx_hbm.at[pl.ds(base, _BPW)], idx_v)

    groups_per_chunk = _CHUNK // _LANES
    for g in range(_BPW // _LANES):
        rep = wid * _REP_PER_TILE + (g // groups_per_chunk) % _REP_PER_TILE
        off = jnp.full((_LANES,), rep * _NUM_TYPES, dtype=jnp.int32)
        idx_v[pl.ds(g * _LANES, _LANES)] = idx_v[pl.ds(g * _LANES, _LANES)] + off

    def src(c):
        return tab_hbm.at[idx_v.at[pl.ds(c * _CHUNK, _CHUNK)]]

    def dst(c):
        return out_hbm.at[pl.ds(base + c * _CHUNK, _CHUNK)]

    def gstart(c):
        pltpu.async_copy(src(c), rows[c % _NBUF], gsem[c % _NBUF])

    def gwait(c):
        pltpu.make_async_copy(src(c), rows[c % _NBUF], gsem[c % _NBUF]).wait()

    def sstart(c):
        pltpu.async_copy(rows[c % _NBUF], dst(c), ssem[c % _NBUF])

    def swait(c):
        pltpu.make_async_copy(rows[c % _NBUF], dst(c), ssem[c % _NBUF]).wait()

    for c in range(_LOOK):
        gstart(c)
    for c in range(_NCHUNK):
        gwait(c)
        sstart(c)
        n = c + _LOOK
        if n < _NCHUNK:
            if n >= _NBUF:
                swait(n - _NBUF)
            gstart(n)
    for c in range(max(0, _NCHUNK - _NBUF), _NCHUNK):
        swait(c)


def _gather(out_table, tok):
    mesh = plsc.VectorSubcoreMesh(
        core_axis_name="c", subcore_axis_name="s", num_cores=_NC
    )
    run = pl.kernel(
        _gather_body,
        out_type=jax.ShapeDtypeStruct((_B, _EMBED), jnp.float32),
        mesh=mesh,
        compiler_params=pltpu.CompilerParams(needs_layout_passes=False),
        scratch_types=(
            [pltpu.VMEM((_BPW,), jnp.int32)]
            + [pltpu.VMEM((_CHUNK, _EMBED), jnp.float32) for _ in range(_NBUF)]
            + [pltpu.SemaphoreType.DMA for _ in range(2 * _NBUF)]
        ),
    )
    return run(out_table, tok)


def kernel(goal_tokens, table, W1, b1, W2, b2):
    tok = goal_tokens.astype(jnp.int32)
    out_table = _mlp_table(table, W1, b1, W2, b2)
    return _gather(out_table, tok)


# speedup vs baseline: 5.9588x; 1.0192x over previous
import jax
import jax.numpy as jnp
from jax import lax
from jax.experimental import pallas as pl
from jax.experimental.pallas import tpu as pltpu
from jax.experimental.pallas import tpu_sc as plsc

_NUM_TYPES = 16
_HIDDEN = 512
_EMBED = 768
_B = 16384

_NC = 2
_NS = 16
_NW = _NC * _NS
_BPW = _B // _NW
_NBUF = 8
_CHUNK = 16
_NCHUNK = _BPW // _CHUNK
_LANES = 16
_REP_PER_TILE = 4
_NREP = _NW * _REP_PER_TILE


def _mlp_body(table_ref, w1_ref, b1_ref, w2_ref, b2_ref, out_ref):
    h = jnp.dot(table_ref[...], w1_ref[...], preferred_element_type=jnp.float32)
    h = h + b1_ref[...]
    h = h * jax.nn.sigmoid(h)
    o = jnp.dot(h, w2_ref[...], preferred_element_type=jnp.float32) + b2_ref[...]
    for r in range(_NREP):
        out_ref[pl.ds(r * _NUM_TYPES, _NUM_TYPES), :] = o


def _mlp_table(table, W1, b1, W2, b2):
    return pl.pallas_call(
        _mlp_body,
        out_shape=jax.ShapeDtypeStruct((_NREP * _NUM_TYPES, _EMBED), jnp.float32),
    )(table, W1, b1.reshape(1, _HIDDEN), W2, b2.reshape(1, _EMBED))


def _gather_body(tab_hbm, idx_hbm, out_hbm, idx_v, *bufs):
    rows = bufs[:_NBUF]
    gsem = bufs[_NBUF : 2 * _NBUF]
    ssem = bufs[2 * _NBUF :]

    wid = lax.axis_index("s") * _NC + lax.axis_index("c")
    base = wid * _BPW
    pltpu.sync_copy(idx_hbm.at[pl.ds(base, _BPW)], idx_v)

    groups_per_chunk = _CHUNK // _LANES
    for g in range(_BPW // _LANES):
        rep = wid * _REP_PER_TILE + (g // groups_per_chunk) % _REP_PER_TILE
        off = jnp.full((_LANES,), rep * _NUM_TYPES, dtype=jnp.int32)
        idx_v[pl.ds(g * _LANES, _LANES)] = idx_v[pl.ds(g * _LANES, _LANES)] + off

    def src(c):
        return tab_hbm.at[idx_v.at[pl.ds(c * _CHUNK, _CHUNK)]]

    def dst(c):
        return out_hbm.at[pl.ds(base + c * _CHUNK, _CHUNK)]

    def gstart(c):
        pltpu.async_copy(src(c), rows[c % _NBUF], gsem[c % _NBUF])

    def gwait(c):
        pltpu.make_async_copy(src(c), rows[c % _NBUF], gsem[c % _NBUF]).wait()

    def sstart(c):
        pltpu.async_copy(rows[c % _NBUF], dst(c), ssem[c % _NBUF])

    def swait(c):
        pltpu.make_async_copy(rows[c % _NBUF], dst(c), ssem[c % _NBUF]).wait()

    for c in range(_NBUF):
        gstart(c)
    for c in range(_NCHUNK):
        gwait(c)
        sstart(c)
        if c + _NBUF < _NCHUNK:
            swait(c)
            gstart(c + _NBUF)
    for c in range(_NCHUNK - _NBUF, _NCHUNK):
        swait(c)


def _gather(out_table, tok):
    mesh = plsc.VectorSubcoreMesh(
        core_axis_name="c", subcore_axis_name="s", num_cores=_NC
    )
    run = pl.kernel(
        _gather_body,
        out_type=jax.ShapeDtypeStruct((_B, _EMBED), jnp.float32),
        mesh=mesh,
        compiler_params=pltpu.CompilerParams(needs_layout_passes=False),
        scratch_types=(
            [pltpu.VMEM((_BPW,), jnp.int32)]
            + [pltpu.VMEM((_CHUNK, _EMBED), jnp.float32) for _ in range(_NBUF)]
            + [pltpu.SemaphoreType.DMA for _ in range(2 * _NBUF)]
        ),
    )
    return run(out_table, tok)


def kernel(goal_tokens, table, W1, b1, W2, b2):
    tok = goal_tokens.astype(jnp.int32)
    out_table = _mlp_table(table, W1, b1, W2, b2)
    return _gather(out_table, tok)
